# baseline (speedup 1.0000x reference)
import functools

import jax
import jax.numpy as jnp
from jax import lax
from jax.experimental import pallas as pl
from jax.experimental.pallas import tpu as pltpu
from jax.experimental.pallas import tpu_sc as plsc

C = 256
K = 15
G = 16
CPG = 16
SIGMA = 1.0
INF = 1000000.0
M = 10000
N = 10000
H = 16
BN_EPS = 1e-5

NC = 2
NS = 16
NW = NC * NS
MP = 10240
MPW = MP // NW
EPW = MPW * H

PCH = 128
PNCH = EPW // PCH

_sc_mesh = plsc.VectorSubcoreMesh(core_axis_name="c", subcore_axis_name="s")


@functools.partial(
    pl.kernel,
    mesh=_sc_mesh,
    out_type=jax.ShapeDtypeStruct((MP * H, 16), jnp.float32),
    scratch_types=[
        pltpu.VMEM((EPW,), jnp.int32),
        pltpu.VMEM((PCH, 16), jnp.float32),
        pltpu.SemaphoreType.DMA,
    ],
    compiler_params=pltpu.CompilerParams(use_tc_tiling_on_sc=False),
)
def _sc_gather_pts(tbl_hbm, idx_hbm, out_hbm, idx_v, rows_v, sem):
    wid = lax.axis_index("s") * NC + lax.axis_index("c")
    base = wid * EPW
    pltpu.sync_copy(idx_hbm.at[pl.ds(base, EPW)], idx_v)

    def step(cc, carry):
        pltpu.async_copy(
            tbl_hbm.at[idx_v.at[pl.ds(cc * PCH, PCH)]], rows_v, sem
        ).wait()
        pltpu.sync_copy(rows_v, out_hbm.at[pl.ds(base + cc * PCH, PCH)])
        return carry

    lax.fori_loop(0, PNCH, step, 0)


NB = 1000


def _tc1_body(sf_ref, wr_ref, br_ref, h_ref, st_ref):
    i = pl.program_id(0)
    h = (
        jnp.dot(sf_ref[...], wr_ref[...], preferred_element_type=jnp.float32)
        + br_ref[...]
    )
    h_ref[...] = h

    @pl.when(i == 0)
    def _():
        st_ref[...] = jnp.zeros_like(st_ref)

    st_ref[...] += jnp.concatenate(
        [
            jnp.sum(h, axis=0, keepdims=True),
            jnp.sum(h * h, axis=0, keepdims=True),
        ],
        axis=0,
    )


def _tc1_call(s_feats, wr_t, br):
    cr = wr_t.shape[1]
    return pl.pallas_call(
        _tc1_body,
        grid=(N // NB,),
        in_specs=[
            pl.BlockSpec((NB, C), lambda i: (i, 0)),
            pl.BlockSpec((C, cr), lambda i: (0, 0)),
            pl.BlockSpec((1, cr), lambda i: (0, 0)),
        ],
        out_specs=[
            pl.BlockSpec((NB, cr), lambda i: (i, 0)),
            pl.BlockSpec((2, cr), lambda i: (0, 0)),
        ],
        out_shape=[
            jax.ShapeDtypeStruct((N, cr), jnp.float32),
            jax.ShapeDtypeStruct((2, cr), jnp.float32),
        ],
    )(s_feats, wr_t, br)


MB = 256


def _tc2_body(h_ref, st_ref, gam_ref, bet_ref, wg_ref, bg_ref, gp_ref,
              qp_ref, kp_ref, a_ref):
    mu = st_ref[0:1, :] * (1.0 / N)
    var = st_ref[1:2, :] * (1.0 / N) - mu * mu
    inv = lax.rsqrt(var + BN_EPS)
    hn = (h_ref[...] - mu) * (inv * gam_ref[...]) + bet_ref[...]
    hn = jnp.where(hn >= 0, hn, 0.1 * hn)
    cw = (
        jnp.dot(hn, wg_ref[...], preferred_element_type=jnp.float32)
        + bg_ref[...]
    )

    d = gp_ref[...] - qp_ref[...][:, None, :]
    acc = jnp.zeros((MB, H, G), jnp.float32)
    for k in range(K):
        v = d - kp_ref[k, :][None, None, :]
        sq = jnp.sum(v * v, axis=2)
        w = jnp.maximum(1.0 - jnp.sqrt(sq) * (1.0 / SIGMA), 0.0)
        cwk = cw[:, k * G:(k + 1) * G]
        acc = acc + w[:, :, None] * cwk[:, None, :]
    a_ref[...] = acc


def _tc2_call(h_pad, st, gam, bet, wg_t, bg, gp3, qp, kp):
    cr = h_pad.shape[1]
    kg = wg_t.shape[1]
    return pl.pallas_call(
        _tc2_body,
        grid=(MP // MB,),
        in_specs=[
            pl.BlockSpec((MB, cr), lambda i: (i, 0)),
            pl.BlockSpec((2, cr), lambda i: (0, 0)),
            pl.BlockSpec((1, cr), lambda i: (0, 0)),
            pl.BlockSpec((1, cr), lambda i: (0, 0)),
            pl.BlockSpec((cr, kg), lambda i: (0, 0)),
            pl.BlockSpec((1, kg), lambda i: (0, 0)),
            pl.BlockSpec((MB, H, 16), lambda i: (i, 0, 0)),
            pl.BlockSpec((MB, 16), lambda i: (i, 0)),
            pl.BlockSpec((16, 16), lambda i: (0, 0)),
        ],
        out_specs=pl.BlockSpec((MB, H, G), lambda i: (i, 0, 0)),
        out_shape=jax.ShapeDtypeStruct((MP, H, G), jnp.float32),
    )(h_pad, st, gam, bet, wg_t, bg, gp3, qp, kp)


MC = 8
RNCH = MPW // MC
RCH = MC * H


@functools.partial(
    pl.kernel,
    mesh=_sc_mesh,
    out_type=jax.ShapeDtypeStruct((MP, C), jnp.float32),
    scratch_types=[
        pltpu.VMEM((EPW,), jnp.int32),
        pltpu.VMEM((MC * C,), jnp.float32),
        pltpu.VMEM((RCH, C), jnp.float32),
        pltpu.VMEM((MC, C), jnp.float32),
        pltpu.SemaphoreType.DMA,
    ],
    compiler_params=pltpu.CompilerParams(use_tc_tiling_on_sc=False),
)
def _sc_reduce(feats_hbm, idx_hbm, a_hbm, out_hbm, idx_v, a_v, rows_v,
               out_v, sem):
    wid = lax.axis_index("s") * NC + lax.axis_index("c")
    mbase = wid * MPW
    pltpu.sync_copy(idx_hbm.at[pl.ds(mbase * H, EPW)], idx_v)

    def chunk(cc, carry):
        pltpu.sync_copy(
            a_hbm.at[pl.ds((mbase + cc * MC) * C, MC * C)], a_v
        )
        pltpu.async_copy(
            feats_hbm.at[idx_v.at[pl.ds(cc * RCH, RCH)]], rows_v, sem
        ).wait()

        def per_m(ml, c2):
            abase = ml * C
            accs = [jnp.zeros((CPG,), jnp.float32) for _ in range(G)]
            for h in range(H):
                av = a_v[pl.ds(abase + h * G, G)]
                for g in range(G):
                    r = rows_v[ml * H + h, pl.ds(g * CPG, CPG)]
                    accs[g] = accs[g] + av[g] * r
            for g in range(G):
                out_v[ml, pl.ds(g * CPG, CPG)] = accs[g]
            return c2

        lax.fori_loop(0, MC, per_m, 0)
        pltpu.sync_copy(out_v, out_hbm.at[pl.ds(mbase + cc * MC, MC)])
        return carry

    lax.fori_loop(0, RNCH, chunk, 0)


def kernel(q_pts, s_pts, s_feats, neighb_inds, kernel_points,
           W_reduce, b_reduce, gamma, beta, W_gen, b_gen):
    idx32 = neighb_inds.astype(jnp.int32)
    idx_pad = jnp.zeros((MP, H), jnp.int32).at[:M].set(idx32)
    flat_idx = idx_pad.reshape(MP * H)

    ptbl = (
        jnp.zeros((N + 1, 16), jnp.float32)
        .at[:N, :3].set(s_pts)
        .at[N, :3].set(INF)
    )
    ftbl = jnp.concatenate(
        [s_feats, jnp.zeros((1, C), jnp.float32)], axis=0
    )
    qp = jnp.zeros((MP, 16), jnp.float32).at[:M, :3].set(q_pts)
    kp = jnp.zeros((16, 16), jnp.float32).at[:K, :3].set(kernel_points)

    gpts = _sc_gather_pts(ptbl, flat_idx)
    h, st = _tc1_call(s_feats, W_reduce.T, b_reduce.reshape(1, -1))
    h_pad = jnp.zeros((MP, h.shape[1]), jnp.float32).at[:M].set(h)
    a3 = _tc2_call(
        h_pad, st, gamma.reshape(1, -1), beta.reshape(1, -1),
        W_gen.T, b_gen.reshape(1, -1),
        gpts.reshape(MP, H, 16), qp, kp,
    )
    out = _sc_reduce(ftbl, flat_idx, a3.reshape(MP * H * G))
    return out[:M]


# speedup vs baseline: 2.4267x; 2.4267x over previous
import functools

import jax
import jax.numpy as jnp
import numpy as np
from jax import lax
from jax.experimental import pallas as pl
from jax.experimental.pallas import tpu as pltpu
from jax.experimental.pallas import tpu_sc as plsc

C = 256
K = 15
G = 16
CPG = 16
SIGMA = 1.0
INF = 1000000.0
M = 10000
N = 10000
H = 16
BN_EPS = 1e-5

NC = 2
NS = 16
NW = NC * NS
MP = 10240
MPW = MP // NW
EPW = MPW * H

_hh = np.arange(H)
_S2 = np.zeros((C, C), np.float32)
for _k in range(16):
    _S2[(_hh[:, None] * 16 + np.arange(16)[None, :]).ravel(),
        np.repeat(_k * 16 + _hh, 16)] = 1.0
_RH = np.zeros((G, C), np.float32)
for _h in range(H):
    _RH[_h, _h * 16 + np.arange(G)] = 1.0
_TG = np.zeros((G, C), np.float32)
for _g in range(G):
    _TG[_g, _hh * 16 + _g] = 1.0

_sc_mesh = plsc.VectorSubcoreMesh(core_axis_name="c", subcore_axis_name="s")

PCH = 128
PNCH = EPW // PCH


@functools.partial(
    pl.kernel,
    mesh=_sc_mesh,
    out_type=jax.ShapeDtypeStruct((MP * H, 16), jnp.float32),
    scratch_types=[
        pltpu.VMEM((EPW,), jnp.int32),
        pltpu.VMEM((PCH, 16), jnp.float32),
        pltpu.VMEM((PCH, 16), jnp.float32),
        pltpu.SemaphoreType.DMA,
        pltpu.SemaphoreType.DMA,
    ],
    compiler_params=pltpu.CompilerParams(use_tc_tiling_on_sc=False),
)
def _sc_gather_pts(tbl_hbm, idx_hbm, out_hbm, idx_v, rows0, rows1, sem0, sem1):
    wid = lax.axis_index("s") * NC + lax.axis_index("c")
    base = wid * EPW
    pltpu.sync_copy(idx_hbm.at[pl.ds(base, EPW)], idx_v)

    bufs = (rows0, rows1)
    sems = (sem0, sem1)

    def issue(cc, b):
        pltpu.async_copy(
            tbl_hbm.at[idx_v.at[pl.ds(cc * PCH, PCH)]], bufs[b], sems[b]
        )

    def drain(b):
        pltpu.make_async_copy(tbl_hbm.at[pl.ds(0, PCH)], bufs[b], sems[b]).wait()

    issue(0, 0)

    def step(j, carry):
        c0 = 2 * j
        drain(0)

        @pl.when(c0 + 1 < PNCH)
        def _():
            issue(c0 + 1, 1)

        pltpu.sync_copy(bufs[0], out_hbm.at[pl.ds(base + c0 * PCH, PCH)])

        @pl.when(c0 + 2 < PNCH)
        def _():
            issue(c0 + 2, 0)

        @pl.when(c0 + 1 < PNCH)
        def _():
            drain(1)
            pltpu.sync_copy(
                bufs[1], out_hbm.at[pl.ds(base + (c0 + 1) * PCH, PCH)]
            )

        return carry

    lax.fori_loop(0, (PNCH + 1) // 2, step, 0)


NB = 1000


def _tc1_body(sf_ref, wr_ref, br_ref, h_ref, st_ref):
    i = pl.program_id(0)
    h = (
        jnp.dot(sf_ref[...], wr_ref[...], preferred_element_type=jnp.float32)
        + br_ref[...]
    )
    h_ref[...] = h

    @pl.when(i == 0)
    def _():
        st_ref[...] = jnp.zeros_like(st_ref)

    st_ref[...] += jnp.concatenate(
        [
            jnp.sum(h, axis=0, keepdims=True),
            jnp.sum(h * h, axis=0, keepdims=True),
        ],
        axis=0,
    )


def _tc1_call(s_feats, wr_t, br):
    cr = wr_t.shape[1]
    return pl.pallas_call(
        _tc1_body,
        grid=(N // NB,),
        in_specs=[
            pl.BlockSpec((NB, C), lambda i: (i, 0)),
            pl.BlockSpec((C, cr), lambda i: (0, 0)),
            pl.BlockSpec((1, cr), lambda i: (0, 0)),
        ],
        out_specs=[
            pl.BlockSpec((NB, cr), lambda i: (i, 0)),
            pl.BlockSpec((2, cr), lambda i: (0, 0)),
        ],
        out_shape=[
            jax.ShapeDtypeStruct((N, cr), jnp.float32),
            jax.ShapeDtypeStruct((2, cr), jnp.float32),
        ],
    )(s_feats, wr_t, br)


MB = 256


def _tc2_body(h_ref, st_ref, gam_ref, bet_ref, wg_ref, bg_ref, d_ref,
              qr_ref, km_ref, kpn_ref, s2_ref, rh_ref, tg_ref, a_ref):
    mu = st_ref[0:1, :] * (1.0 / N)
    var = st_ref[1:2, :] * (1.0 / N) - mu * mu
    inv = lax.rsqrt(var + BN_EPS)
    hn = (h_ref[...] - mu) * (inv * gam_ref[...]) + bet_ref[...]
    hn = jnp.where(hn >= 0, hn, 0.1 * hn)
    cw = (
        jnp.dot(hn, wg_ref[...], preferred_element_type=jnp.float32)
        + bg_ref[...]
    )

    d0 = d_ref[...] - qr_ref[...]
    n0e = jnp.dot(d0 * d0, s2_ref[...],
                  preferred_element_type=jnp.float32)
    dkp = jnp.dot(d0, km_ref[...],
                  preferred_element_type=jnp.float32)
    sq = n0e - 2.0 * dkp + kpn_ref[...]
    w2 = jnp.maximum(1.0 - jnp.sqrt(sq) * (1.0 / SIGMA), 0.0)

    acc = jnp.zeros((MB, C), jnp.float32)
    for k in range(K):
        wk = w2[:, k * 16:(k + 1) * 16]
        cwk = cw[:, k * G:(k + 1) * G]
        wexp = jnp.dot(wk, rh_ref[...],
                       preferred_element_type=jnp.float32)
        cwexp = jnp.dot(cwk, tg_ref[...],
                        preferred_element_type=jnp.float32)
        acc = acc + wexp * cwexp
    a_ref[...] = acc


def _tc2_call(h_pad, st, gam, bet, wg_t, bg, d_in, qrep, km, kpn, s2, rh, tg):
    cr = h_pad.shape[1]
    kg = wg_t.shape[1]
    return pl.pallas_call(
        _tc2_body,
        grid=(MP // MB,),
        in_specs=[
            pl.BlockSpec((MB, cr), lambda i: (i, 0)),
            pl.BlockSpec((2, cr), lambda i: (0, 0)),
            pl.BlockSpec((1, cr), lambda i: (0, 0)),
            pl.BlockSpec((1, cr), lambda i: (0, 0)),
            pl.BlockSpec((cr, kg), lambda i: (0, 0)),
            pl.BlockSpec((1, kg), lambda i: (0, 0)),
            pl.BlockSpec((MB, C), lambda i: (i, 0)),
            pl.BlockSpec((MB, C), lambda i: (i, 0)),
            pl.BlockSpec((C, C), lambda i: (0, 0)),
            pl.BlockSpec((1, C), lambda i: (0, 0)),
            pl.BlockSpec((C, C), lambda i: (0, 0)),
            pl.BlockSpec((G, C), lambda i: (0, 0)),
            pl.BlockSpec((G, C), lambda i: (0, 0)),
        ],
        out_specs=pl.BlockSpec((MB, C), lambda i: (i, 0)),
        out_shape=jax.ShapeDtypeStruct((MP, C), jnp.float32),
    )(h_pad, st, gam, bet, wg_t, bg, d_in, qrep, km, kpn, s2, rh, tg)


MC = 8
RNCH = MPW // MC
RCH = MC * H


@functools.partial(
    pl.kernel,
    mesh=_sc_mesh,
    out_type=jax.ShapeDtypeStruct((MP, C), jnp.float32),
    scratch_types=[
        pltpu.VMEM((EPW,), jnp.int32),
        pltpu.VMEM((MC * C,), jnp.float32),
        pltpu.VMEM((MC * C,), jnp.float32),
        pltpu.VMEM((RCH, C), jnp.float32),
        pltpu.VMEM((RCH, C), jnp.float32),
        pltpu.VMEM((MC, C), jnp.float32),
        pltpu.SemaphoreType.DMA,
        pltpu.SemaphoreType.DMA,
    ],
    compiler_params=pltpu.CompilerParams(use_tc_tiling_on_sc=False),
)
def _sc_reduce(feats_hbm, idx_hbm, a_hbm, out_hbm, idx_v, a0, a1,
               rows0, rows1, out_v, sem0, sem1):
    wid = lax.axis_index("s") * NC + lax.axis_index("c")
    mbase = wid * MPW
    pltpu.sync_copy(idx_hbm.at[pl.ds(mbase * H, EPW)], idx_v)

    abufs = (a0, a1)
    rbufs = (rows0, rows1)
    sems = (sem0, sem1)

    def issue(cc, b):
        pltpu.async_copy(
            feats_hbm.at[idx_v.at[pl.ds(cc * RCH, RCH)]], rbufs[b], sems[b]
        )
        pltpu.async_copy(
            a_hbm.at[pl.ds((mbase + cc * MC) * C, MC * C)], abufs[b], sems[b]
        )

    def drain(b):
        pltpu.make_async_copy(
            feats_hbm.at[pl.ds(0, RCH)], rbufs[b], sems[b]
        ).wait()
        pltpu.make_async_copy(
            a_hbm.at[pl.ds(0, MC * C)], abufs[b], sems[b]
        ).wait()

    def compute(cc, b):
        a_v = abufs[b]
        rows_v = rbufs[b]

        def per_m(ml, c2):
            abase = ml * C
            accs = [jnp.zeros((CPG,), jnp.float32) for _ in range(G)]
            for h in range(H):
                av = a_v[pl.ds(abase + h * G, G)]
                for g in range(G):
                    r = rows_v[ml * H + h, pl.ds(g * CPG, CPG)]
                    accs[g] = accs[g] + av[g] * r
            for g in range(G):
                out_v[ml, pl.ds(g * CPG, CPG)] = accs[g]
            return c2

        lax.fori_loop(0, MC, per_m, 0)
        pltpu.sync_copy(out_v, out_hbm.at[pl.ds(mbase + cc * MC, MC)])

    issue(0, 0)

    def step(j, carry):
        c0 = 2 * j
        drain(0)

        @pl.when(c0 + 1 < RNCH)
        def _():
            issue(c0 + 1, 1)

        compute(c0, 0)

        @pl.when(c0 + 2 < RNCH)
        def _():
            issue(c0 + 2, 0)

        @pl.when(c0 + 1 < RNCH)
        def _():
            drain(1)
            compute(c0 + 1, 1)

        return carry

    lax.fori_loop(0, (RNCH + 1) // 2, step, 0)


def kernel(q_pts, s_pts, s_feats, neighb_inds, kernel_points,
           W_reduce, b_reduce, gamma, beta, W_gen, b_gen):
    idx32 = neighb_inds.astype(jnp.int32)
    idx_pad = jnp.zeros((MP, H), jnp.int32).at[:M].set(idx32)
    flat_idx = idx_pad.reshape(MP * H)

    ptbl = (
        jnp.zeros((N + 1, 16), jnp.float32)
        .at[:N, :3].set(s_pts)
        .at[N, :3].set(INF)
    )
    ftbl = jnp.concatenate(
        [s_feats, jnp.zeros((1, C), jnp.float32)], axis=0
    )
    qp = jnp.zeros((MP, 16), jnp.float32).at[:M, :3].set(q_pts)
    qrep = jnp.tile(qp, (1, H))

    kk, cc2, hh = np.meshgrid(np.arange(K), np.arange(3), np.arange(H),
                              indexing="ij")
    km = (
        jnp.zeros((C, C), jnp.float32)
        .at[(hh * 16 + cc2).ravel(), (kk * 16 + hh).ravel()]
        .set(jnp.broadcast_to(kernel_points[:, :, None], (K, 3, H)).ravel())
    )
    kn = jnp.sum(kernel_points * kernel_points, axis=1)
    kk2, hh2 = np.meshgrid(np.arange(K), np.arange(H), indexing="ij")
    kpn = (
        jnp.zeros((C,), jnp.float32)
        .at[(kk2 * 16 + hh2).ravel()]
        .set(jnp.broadcast_to(kn[:, None], (K, H)).ravel())
        .reshape(1, C)
    )

    gpts = _sc_gather_pts(ptbl, flat_idx)
    h, st = _tc1_call(s_feats, W_reduce.T, b_reduce.reshape(1, -1))
    h_pad = jnp.zeros((MP, h.shape[1]), jnp.float32).at[:M].set(h)
    a2 = _tc2_call(
        h_pad, st, gamma.reshape(1, -1), beta.reshape(1, -1),
        W_gen.T, b_gen.reshape(1, -1),
        gpts.reshape(MP, C), qrep, km, kpn,
        jnp.asarray(_S2), jnp.asarray(_RH), jnp.asarray(_TG),
    )
    out = _sc_reduce(ftbl, flat_idx, a2.reshape(MP * C))
    return out[:M]


# speedup vs baseline: 2.7764x; 1.1441x over previous
import functools

import jax
import jax.numpy as jnp
import numpy as np
from jax import lax
from jax.experimental import pallas as pl
from jax.experimental.pallas import tpu as pltpu
from jax.experimental.pallas import tpu_sc as plsc

C = 256
K = 15
G = 16
CPG = 16
SIGMA = 1.0
INF = 1000000.0
M = 10000
N = 10000
H = 16
BN_EPS = 1e-5

NC = 2
NS = 16
NW = NC * NS
MP = 10240
MPW = MP // NW
EPW = MPW * H

_hh = np.arange(H)
_S2 = np.zeros((C, C), np.float32)
for _k in range(16):
    _S2[(_hh[:, None] * 16 + np.arange(16)[None, :]).ravel(),
        np.repeat(_k * 16 + _hh, 16)] = 1.0
_RH = np.zeros((G, C), np.float32)
for _h in range(H):
    _RH[_h, _h * 16 + np.arange(G)] = 1.0
_TG = np.zeros((G, C), np.float32)
for _g in range(G):
    _TG[_g, _hh * 16 + _g] = 1.0
_QT = np.zeros((16, C), np.float32)
for _c in range(16):
    _QT[_c, _hh * 16 + _c] = 1.0
_KMASK = np.zeros((C, C), np.float32)
for _h in range(H):
    for _c in range(16):
        for _k in range(16):
            _KMASK[_h * 16 + _c, _k * 16 + _h] = 1.0

_sc_mesh = plsc.VectorSubcoreMesh(core_axis_name="c", subcore_axis_name="s")

PCH = 128
PNCH = EPW // PCH


@functools.partial(
    pl.kernel,
    mesh=_sc_mesh,
    out_type=jax.ShapeDtypeStruct((MP * H, 16), jnp.float32),
    scratch_types=[
        pltpu.VMEM((EPW,), jnp.int32),
        pltpu.VMEM((PCH, 16), jnp.float32),
        pltpu.VMEM((PCH, 16), jnp.float32),
        pltpu.SemaphoreType.DMA,
        pltpu.SemaphoreType.DMA,
    ],
    compiler_params=pltpu.CompilerParams(use_tc_tiling_on_sc=False),
)
def _sc_gather_pts(tbl_hbm, idx_hbm, out_hbm, idx_v, rows0, rows1, sem0, sem1):
    wid = lax.axis_index("s") * NC + lax.axis_index("c")
    base = wid * EPW
    pltpu.sync_copy(idx_hbm.at[pl.ds(base, EPW)], idx_v)

    bufs = (rows0, rows1)
    sems = (sem0, sem1)

    def issue(cc, b):
        pltpu.async_copy(
            tbl_hbm.at[idx_v.at[pl.ds(cc * PCH, PCH)]], bufs[b], sems[b]
        )

    def drain(b):
        pltpu.make_async_copy(tbl_hbm.at[pl.ds(0, PCH)], bufs[b], sems[b]).wait()

    issue(0, 0)

    def step(j, carry):
        c0 = 2 * j
        drain(0)

        @pl.when(c0 + 1 < PNCH)
        def _():
            issue(c0 + 1, 1)

        pltpu.sync_copy(bufs[0], out_hbm.at[pl.ds(base + c0 * PCH, PCH)])

        @pl.when(c0 + 2 < PNCH)
        def _():
            issue(c0 + 2, 0)

        @pl.when(c0 + 1 < PNCH)
        def _():
            drain(1)
            pltpu.sync_copy(
                bufs[1], out_hbm.at[pl.ds(base + (c0 + 1) * PCH, PCH)]
            )

        return carry

    lax.fori_loop(0, (PNCH + 1) // 2, step, 0)


NB = 1000


def _tc1_body(sf_ref, wr_ref, br_ref, h_ref, st_ref):
    i = pl.program_id(0)
    h = (
        jnp.dot(sf_ref[...], wr_ref[...], preferred_element_type=jnp.float32)
        + br_ref[...]
    )
    h_ref[...] = h

    @pl.when(i == 0)
    def _():
        st_ref[...] = jnp.zeros_like(st_ref)

    st_ref[...] += jnp.concatenate(
        [
            jnp.sum(h, axis=0, keepdims=True),
            jnp.sum(h * h, axis=0, keepdims=True),
        ],
        axis=0,
    )


def _tc1_call(s_feats, wr_t, br):
    cr = wr_t.shape[1]
    return pl.pallas_call(
        _tc1_body,
        grid=(N // NB,),
        in_specs=[
            pl.BlockSpec((NB, C), lambda i: (i, 0)),
            pl.BlockSpec((C, cr), lambda i: (0, 0)),
            pl.BlockSpec((1, cr), lambda i: (0, 0)),
        ],
        out_specs=[
            pl.BlockSpec((NB, cr), lambda i: (i, 0)),
            pl.BlockSpec((2, cr), lambda i: (0, 0)),
        ],
        out_shape=[
            jax.ShapeDtypeStruct((N, cr), jnp.float32),
            jax.ShapeDtypeStruct((2, cr), jnp.float32),
        ],
    )(s_feats, wr_t, br)


MB = 256


def _tc2_body(h_ref, st_ref, gam_ref, bet_ref, wg_ref, bg_ref, d_ref,
              qp_ref, qt_ref, km_ref, kpn_ref, s2_ref, rh_ref, tg_ref,
              a_ref):
    mu = st_ref[0:1, :] * (1.0 / N)
    var = st_ref[1:2, :] * (1.0 / N) - mu * mu
    inv = lax.rsqrt(var + BN_EPS)
    hn = (h_ref[...] - mu) * (inv * gam_ref[...]) + bet_ref[...]
    hn = jnp.where(hn >= 0, hn, 0.1 * hn)
    cw = (
        jnp.dot(hn, wg_ref[...], preferred_element_type=jnp.float32)
        + bg_ref[...]
    )

    qrep = jnp.dot(qp_ref[...], qt_ref[...],
                   preferred_element_type=jnp.float32)
    d0 = d_ref[...] - qrep
    n0e = jnp.dot(d0 * d0, s2_ref[...],
                  preferred_element_type=jnp.float32)
    dkp = jnp.dot(d0, km_ref[...],
                  preferred_element_type=jnp.float32)
    sq = n0e - 2.0 * dkp + kpn_ref[...]
    w2 = jnp.maximum(1.0 - jnp.sqrt(sq) * (1.0 / SIGMA), 0.0)

    acc = jnp.zeros((MB, C), jnp.float32)
    for k in range(K):
        wk = w2[:, k * 16:(k + 1) * 16]
        cwk = cw[:, k * G:(k + 1) * G]
        wexp = jnp.dot(wk, rh_ref[...],
                       preferred_element_type=jnp.float32)
        cwexp = jnp.dot(cwk, tg_ref[...],
                        preferred_element_type=jnp.float32)
        acc = acc + wexp * cwexp
    a_ref[...] = acc


def _tc2_call(h_pad, st, gam, bet, wg_t, bg, d_in, qp, qt, km, kpn, s2,
              rh, tg):
    cr = h_pad.shape[1]
    kg = wg_t.shape[1]
    return pl.pallas_call(
        _tc2_body,
        grid=(MP // MB,),
        in_specs=[
            pl.BlockSpec((MB, cr), lambda i: (i, 0)),
            pl.BlockSpec((2, cr), lambda i: (0, 0)),
            pl.BlockSpec((1, cr), lambda i: (0, 0)),
            pl.BlockSpec((1, cr), lambda i: (0, 0)),
            pl.BlockSpec((cr, kg), lambda i: (0, 0)),
            pl.BlockSpec((1, kg), lambda i: (0, 0)),
            pl.BlockSpec((MB, C), lambda i: (i, 0)),
            pl.BlockSpec((MB, 16), lambda i: (i, 0)),
            pl.BlockSpec((16, C), lambda i: (0, 0)),
            pl.BlockSpec((C, C), lambda i: (0, 0)),
            pl.BlockSpec((1, C), lambda i: (0, 0)),
            pl.BlockSpec((C, C), lambda i: (0, 0)),
            pl.BlockSpec((G, C), lambda i: (0, 0)),
            pl.BlockSpec((G, C), lambda i: (0, 0)),
        ],
        out_specs=pl.BlockSpec((MB, C), lambda i: (i, 0)),
        out_shape=jax.ShapeDtypeStruct((MP, C), jnp.float32),
    )(h_pad, st, gam, bet, wg_t, bg, d_in, qp, qt, km, kpn, s2, rh, tg)


MC = 8
RNCH = MPW // MC
RCH = MC * H


@functools.partial(
    pl.kernel,
    mesh=_sc_mesh,
    out_type=jax.ShapeDtypeStruct((M, C), jnp.float32),
    scratch_types=[
        pltpu.VMEM((EPW,), jnp.int32),
        pltpu.VMEM((MC * C,), jnp.float32),
        pltpu.VMEM((MC * C,), jnp.float32),
        pltpu.VMEM((RCH, C), jnp.float32),
        pltpu.VMEM((RCH, C), jnp.float32),
        pltpu.VMEM((MC, C), jnp.float32),
        pltpu.SemaphoreType.DMA,
        pltpu.SemaphoreType.DMA,
    ],
    compiler_params=pltpu.CompilerParams(use_tc_tiling_on_sc=False),
)
def _sc_reduce(feats_hbm, idx_hbm, a_hbm, out_hbm, idx_v, a0, a1,
               rows0, rows1, out_v, sem0, sem1):
    wid = lax.axis_index("s") * NC + lax.axis_index("c")
    mbase = wid * MPW
    pltpu.sync_copy(idx_hbm.at[pl.ds(mbase * H, EPW)], idx_v)

    abufs = (a0, a1)
    rbufs = (rows0, rows1)
    sems = (sem0, sem1)

    def issue(cc, b):
        pltpu.async_copy(
            feats_hbm.at[idx_v.at[pl.ds(cc * RCH, RCH)]], rbufs[b], sems[b]
        )
        pltpu.async_copy(
            a_hbm.at[pl.ds((mbase + cc * MC) * C, MC * C)], abufs[b], sems[b]
        )

    def drain(b):
        pltpu.make_async_copy(
            feats_hbm.at[pl.ds(0, RCH)], rbufs[b], sems[b]
        ).wait()
        pltpu.make_async_copy(
            a_hbm.at[pl.ds(0, MC * C)], abufs[b], sems[b]
        ).wait()

    def compute(cc, b):
        a_v = abufs[b]
        rows_v = rbufs[b]

        def per_m(ml, c2):
            abase = ml * C
            accs = [jnp.zeros((CPG,), jnp.float32) for _ in range(G)]
            for h in range(H):
                av = a_v[pl.ds(abase + h * G, G)]
                for g in range(G):
                    r = rows_v[ml * H + h, pl.ds(g * CPG, CPG)]
                    accs[g] = accs[g] + av[g] * r
            for g in range(G):
                out_v[ml, pl.ds(g * CPG, CPG)] = accs[g]
            return c2

        lax.fori_loop(0, MC, per_m, 0)

        @pl.when(mbase + cc * MC + MC <= M)
        def _():
            pltpu.sync_copy(out_v, out_hbm.at[pl.ds(mbase + cc * MC, MC)])

    issue(0, 0)

    def step(j, carry):
        c0 = 2 * j
        drain(0)

        @pl.when(c0 + 1 < RNCH)
        def _():
            issue(c0 + 1, 1)

        compute(c0, 0)

        @pl.when(c0 + 2 < RNCH)
        def _():
            issue(c0 + 2, 0)

        @pl.when(c0 + 1 < RNCH)
        def _():
            drain(1)
            compute(c0 + 1, 1)

        return carry

    lax.fori_loop(0, (RNCH + 1) // 2, step, 0)


def kernel(q_pts, s_pts, s_feats, neighb_inds, kernel_points,
           W_reduce, b_reduce, gamma, beta, W_gen, b_gen):
    idx32 = neighb_inds.astype(jnp.int32)
    idx_pad = jnp.zeros((MP, H), jnp.int32).at[:M].set(idx32)
    flat_idx = idx_pad.reshape(MP * H)

    ptbl = (
        jnp.zeros((N + 1, 16), jnp.float32)
        .at[:N, :3].set(s_pts)
        .at[N, :3].set(INF)
    )
    ftbl = jnp.concatenate(
        [s_feats, jnp.zeros((1, C), jnp.float32)], axis=0
    )
    qp = jnp.zeros((MP, 16), jnp.float32).at[:M, :3].set(q_pts)

    kpT = jnp.zeros((16, 16), jnp.float32).at[:3, :K].set(kernel_points.T)
    km = jnp.asarray(_KMASK) * jnp.tile(jnp.repeat(kpT, 16, axis=1), (16, 1))
    kn16 = (
        jnp.zeros((16,), jnp.float32)
        .at[:K].set(jnp.sum(kernel_points * kernel_points, axis=1))
    )
    kpn = jnp.repeat(kn16, 16).reshape(1, C)

    gpts = _sc_gather_pts(ptbl, flat_idx)
    h, st = _tc1_call(s_feats, W_reduce.T, b_reduce.reshape(1, -1))
    h_pad = jnp.zeros((MP, h.shape[1]), jnp.float32).at[:M].set(h)
    a2 = _tc2_call(
        h_pad, st, gamma.reshape(1, -1), beta.reshape(1, -1),
        W_gen.T, b_gen.reshape(1, -1),
        gpts.reshape(MP, C), qp, jnp.asarray(_QT), km, kpn,
        jnp.asarray(_S2), jnp.asarray(_RH), jnp.asarray(_TG),
    )
    return _sc_reduce(ftbl, flat_idx, a2.reshape(MP * C))
